# baseline (speedup 1.0000x reference)
import functools

import jax
import jax.numpy as jnp
from jax.experimental import pallas as pl


def _ffn_body(x_ref, comb_ref, gw_ref, uw_ref, dw_ref, out_ref, *, n_experts):
    e = pl.program_id(0)
    i = pl.program_id(1)

    @pl.when((e == 0) & (i == 0))
    def _init():
        out_ref[...] = jnp.zeros_like(out_ref)

    xb = x_ref[...]
    gw = gw_ref[0].astype(jnp.bfloat16)
    uw = uw_ref[0].astype(jnp.bfloat16)
    dw = dw_ref[0].astype(jnp.bfloat16)

    dn = (((1,), (1,)), ((), ()))
    g = jax.lax.dot_general(xb, gw, dn, preferred_element_type=jnp.float32)
    u = jax.lax.dot_general(xb, uw, dn, preferred_element_type=jnp.float32)
    h = jax.nn.silu(g) * u

    col = jax.lax.broadcasted_iota(jnp.int32, comb_ref.shape, 1)
    c = jnp.sum(jnp.where(col == e, comb_ref[...], 0.0), axis=1, keepdims=True)
    h = (h * c).astype(jnp.bfloat16)

    contrib = jax.lax.dot_general(h, dw, dn, preferred_element_type=jnp.float32)
    out_ref[...] += contrib


def kernel(x, router_w, router_b, gate_w, up_w, down_w):
    B, S, D = x.shape
    E, DI, _ = gate_w.shape
    N = B * S
    xf = x.reshape(N, D)

    logits = xf @ router_w.T + router_b
    probs = jax.nn.softmax(logits, axis=-1)
    topk_p, topk_i = jax.lax.top_k(probs, 2)
    topk_p = topk_p / jnp.sum(topk_p, axis=-1, keepdims=True)
    combine = jnp.sum(
        topk_p[..., None] * jax.nn.one_hot(topk_i, E, dtype=x.dtype), axis=-2
    )

    Ki = min(256, DI)
    NI = DI // Ki
    x_bf = xf.astype(jnp.bfloat16)

    out = pl.pallas_call(
        functools.partial(_ffn_body, n_experts=E),
        grid=(E, NI),
        in_specs=[
            pl.BlockSpec((N, D), lambda e, i: (0, 0)),
            pl.BlockSpec((N, E), lambda e, i: (0, 0)),
            pl.BlockSpec((1, Ki, D), lambda e, i: (e, i, 0)),
            pl.BlockSpec((1, Ki, D), lambda e, i: (e, i, 0)),
            pl.BlockSpec((1, D, Ki), lambda e, i: (e, 0, i)),
        ],
        out_specs=pl.BlockSpec((N, D), lambda e, i: (0, 0)),
        out_shape=jax.ShapeDtypeStruct((N, D), jnp.float32),
    )(x_bf, combine, gate_w, up_w, down_w)

    return out.reshape(B, S, D)


# speedup vs baseline: 1.0803x; 1.0803x over previous
import functools

import jax
import jax.numpy as jnp
from jax.experimental import pallas as pl
from jax.experimental.pallas import tpu as pltpu

_TOPK = 2


def _grouped_ffn_body(te_ref, xg_ref, gw_ref, uw_ref, dw_ref, out_ref, *, m):
    i = pl.program_id(0)
    t = pl.program_id(1)

    @pl.when((i == 0) & (t == 0))
    def _init():
        out_ref[...] = jnp.zeros_like(out_ref)

    xb = xg_ref[...]
    gw = gw_ref[0].astype(jnp.bfloat16)
    uw = uw_ref[0].astype(jnp.bfloat16)
    dw = dw_ref[0].astype(jnp.bfloat16)

    dn = (((1,), (1,)), ((), ()))
    g = jax.lax.dot_general(xb, gw, dn, preferred_element_type=jnp.float32)
    u = jax.lax.dot_general(xb, uw, dn, preferred_element_type=jnp.float32)
    h = (jax.nn.silu(g) * u).astype(jnp.bfloat16)

    contrib = jax.lax.dot_general(h, dw, dn, preferred_element_type=jnp.float32)
    out_ref[pl.ds(t * m, m), :] += contrib


def kernel(x, router_w, router_b, gate_w, up_w, down_w):
    B, S, D = x.shape
    E, DI, _ = gate_w.shape
    N = B * S
    P = N * _TOPK
    xf = x.reshape(N, D)

    logits = xf @ router_w.T + router_b
    probs = jax.nn.softmax(logits, axis=-1)
    topk_p, topk_i = jax.lax.top_k(probs, _TOPK)
    topk_p = topk_p / jnp.sum(topk_p, axis=-1, keepdims=True)

    ei = topk_i.reshape(P).astype(jnp.int32)
    wv = topk_p.reshape(P)
    tok = (jnp.arange(P, dtype=jnp.int32) // _TOPK)

    M = 256
    T = P // M + E
    PP = T * M
    order = jnp.argsort(ei)
    se = ei[order]
    stok = tok[order]
    counts = jnp.bincount(ei, length=E)
    padded = ((counts + M - 1) // M) * M
    gstart = jnp.concatenate([jnp.zeros((1,), counts.dtype), jnp.cumsum(counts)[:-1]])
    pstart = jnp.concatenate([jnp.zeros((1,), padded.dtype), jnp.cumsum(padded)[:-1]])
    rank = jnp.arange(P, dtype=jnp.int32) - gstart[se].astype(jnp.int32)
    pos = pstart[se].astype(jnp.int32) + rank

    tokp = jnp.zeros((PP,), jnp.int32).at[pos].set(stok)
    pend = jnp.cumsum(padded)
    tile_start = jnp.arange(T, dtype=pend.dtype) * M
    te = jnp.minimum(
        jnp.searchsorted(pend, tile_start, side="right"), E - 1
    ).astype(jnp.int32)

    xg = jnp.take(xf.astype(jnp.bfloat16), tokp, axis=0)

    Ki = min(512, DI)
    NI = DI // Ki

    grid_spec = pltpu.PrefetchScalarGridSpec(
        num_scalar_prefetch=1,
        grid=(NI, T),
        in_specs=[
            pl.BlockSpec((M, D), lambda i, t, te_r: (t, 0)),
            pl.BlockSpec((1, Ki, D), lambda i, t, te_r: (te_r[t], i, 0)),
            pl.BlockSpec((1, Ki, D), lambda i, t, te_r: (te_r[t], i, 0)),
            pl.BlockSpec((1, D, Ki), lambda i, t, te_r: (te_r[t], 0, i)),
        ],
        out_specs=pl.BlockSpec((PP, D), lambda i, t, te_r: (0, 0)),
    )

    yg = pl.pallas_call(
        functools.partial(_grouped_ffn_body, m=M),
        grid_spec=grid_spec,
        out_shape=jax.ShapeDtypeStruct((PP, D), jnp.float32),
    )(te, xg, gate_w, up_w, down_w)

    posp = jnp.zeros((P,), jnp.int32).at[order].set(pos)
    y = (wv[:, None] * yg[posp]).reshape(N, _TOPK, D).sum(axis=1)
    return y.reshape(B, S, D)


# speedup vs baseline: 1.1466x; 1.0613x over previous
import functools

import jax
import jax.numpy as jnp
from jax.experimental import pallas as pl
from jax.experimental.pallas import tpu as pltpu

_TOPK = 2


def _grouped_ffn_body(tile_r, lo_r, hi_r, te_r, xg_ref, gw_ref, uw_ref, dw_ref,
                      out_ref, *, m):
    i = pl.program_id(0)
    s = pl.program_id(1)
    tile = tile_r[s]
    lo = lo_r[s]
    hi = hi_r[s]

    xb = xg_ref[pl.ds(tile * m, m), :].astype(jnp.float32)
    gw = gw_ref[0]
    uw = uw_ref[0]
    dw = dw_ref[0]

    dn = (((1,), (1,)), ((), ()))
    g = jax.lax.dot_general(xb, gw, dn, preferred_element_type=jnp.float32)
    u = jax.lax.dot_general(xb, uw, dn, preferred_element_type=jnp.float32)
    h = jax.nn.silu(g) * u

    contrib = jax.lax.dot_general(h, dw, dn, preferred_element_type=jnp.float32)

    row = jax.lax.broadcasted_iota(jnp.int32, (m, 1), 0)
    contrib = jnp.where((row >= lo) & (row < hi), contrib, 0.0)

    @pl.when((i == 0) & (lo == 0))
    def _set():
        out_ref[pl.ds(tile * m, m), :] = contrib

    @pl.when((i > 0) | (lo > 0))
    def _acc():
        out_ref[pl.ds(tile * m, m), :] += contrib


def kernel(x, router_w, router_b, gate_w, up_w, down_w):
    B, S, D = x.shape
    E, DI, _ = gate_w.shape
    N = B * S
    P = N * _TOPK
    xf = x.reshape(N, D)

    logits = xf @ router_w.T + router_b
    probs = jax.nn.softmax(logits, axis=-1)
    topk_p, topk_i = jax.lax.top_k(probs, _TOPK)
    topk_p = topk_p / jnp.sum(topk_p, axis=-1, keepdims=True)

    ei = topk_i.reshape(P).astype(jnp.int32)
    wv = topk_p.reshape(P)
    tok = (jnp.arange(P, dtype=jnp.int32) // _TOPK)

    M = 256
    NSEG = P // M + E
    order = jnp.argsort(ei)
    stok = tok[order]
    counts = jnp.bincount(ei, length=E)
    gend = jnp.cumsum(counts).astype(jnp.int32)
    gstart = jnp.concatenate([jnp.zeros((1,), jnp.int32), gend[:-1]])

    bps = jnp.sort(
        jnp.concatenate([jnp.arange(P // M, dtype=jnp.int32) * M, gstart])
    )
    ends = jnp.concatenate([bps[1:], jnp.full((1,), P, jnp.int32)])
    seg_tile = bps // M
    seg_lo = bps - seg_tile * M
    seg_hi = ends - seg_tile * M
    seg_te = jnp.minimum(
        jnp.searchsorted(gend, bps, side="right"), E - 1
    ).astype(jnp.int32)

    xg = jnp.take(xf.astype(jnp.bfloat16), stok, axis=0)

    Ki = min(512, DI)
    NI = DI // Ki

    grid_spec = pltpu.PrefetchScalarGridSpec(
        num_scalar_prefetch=4,
        grid=(NI, NSEG),
        in_specs=[
            pl.BlockSpec((P, D), lambda i, s, t_r, l_r, h_r, e_r: (0, 0)),
            pl.BlockSpec((1, Ki, D), lambda i, s, t_r, l_r, h_r, e_r: (e_r[s], i, 0)),
            pl.BlockSpec((1, Ki, D), lambda i, s, t_r, l_r, h_r, e_r: (e_r[s], i, 0)),
            pl.BlockSpec((1, D, Ki), lambda i, s, t_r, l_r, h_r, e_r: (e_r[s], 0, i)),
        ],
        out_specs=pl.BlockSpec((P, D), lambda i, s, t_r, l_r, h_r, e_r: (0, 0)),
    )

    yg = pl.pallas_call(
        functools.partial(_grouped_ffn_body, m=M),
        grid_spec=grid_spec,
        out_shape=jax.ShapeDtypeStruct((P, D), jnp.float32),
        compiler_params=pltpu.CompilerParams(vmem_limit_bytes=67108864),
    )(seg_tile, seg_lo, seg_hi, seg_te, xg, gate_w, up_w, down_w)

    posp = jnp.zeros((P,), jnp.int32).at[order].set(jnp.arange(P, dtype=jnp.int32))
    y = (wv[:, None] * yg[posp]).reshape(N, _TOPK, D).sum(axis=1)
    return y.reshape(B, S, D)
